# baseline (speedup 1.0000x reference)
import functools
import math

import jax
import jax.numpy as jnp
from jax import lax
from jax.experimental import pallas as pl
from jax.experimental.pallas import tpu as pltpu
from jax.experimental.pallas import tpu_sc as plsc

VOCAB = 50265
D = 768
LANES = 16
DJ = D // LANES
NC = 2
NS = 16
NW = NC * NS
INV_SQRT_D = 1.0 / math.sqrt(D)
NCLS = 9

N = 4 * 4096
PER_W = N // NW
C = 32
NBUF = 4
NCHUNK = PER_W // C


def _body(widx_hbm, tidx_hbm, sidx_hbm, wtab_hbm, ttab_hbm, stab_hbm, out_hbm,
          widx_all, cidx_all, tvec_all, svec_all, gidx_all, oidx2d, cnts1d,
          tt_v, st_v, comb_v, rows_v, gsems, osems):
    wid = lax.axis_index("s") * NC + lax.axis_index("c")
    base = wid * PER_W
    iota = lax.iota(jnp.int32, LANES)

    pltpu.sync_copy(widx_hbm.at[pl.ds(base, PER_W)], widx_all)
    pltpu.sync_copy(tidx_hbm.at[pl.ds(base, PER_W)], tvec_all)
    pltpu.sync_copy(sidx_hbm.at[pl.ds(base, PER_W)], svec_all)

    @plsc.parallel_loop(0, PER_W // LANES, unroll=4)
    def _(j):
        sl = pl.ds(j * LANES, LANES)
        cidx_all[sl] = tvec_all[sl] * 3 + svec_all[sl]

    def group_body(g, carry):
        goff = g * C
        gvec = jnp.full((LANES,), g, jnp.int32)
        k0 = cidx_all[pl.ds(goff, LANES)] * C + iota
        k1 = cidx_all[pl.ds(goff + LANES, LANES)] * C + (iota + LANES)
        s0 = jnp.sort(k0)
        r1 = lax.rev(jnp.sort(k1), (0,))
        halves = (jnp.sort(jnp.minimum(s0, r1)),
                  jnp.sort(jnp.maximum(s0, r1)))
        end = jnp.int32(0)
        endv = jnp.zeros((LANES,), jnp.int32)
        for k in range(NCLS):
            nk = jnp.int32(0)
            for sv in halves:
                eq = lax.shift_right_logical(sv, 5) == k
                nk = nk + plsc.all_reduce_population_count(eq)[0]
            end = end + nk
            endv = endv + jnp.where(iota == k, end, 0)
        cnts1d[pl.ds(g * LANES, LANES)] = endv
        for half in range(2):
            rid = lax.bitwise_and(halves[half], C - 1)
            wv = plsc.load_gather(widx_all, [goff + rid])
            gidx_all[pl.ds(goff + half * LANES, LANES)] = wv
            orow = base + goff + rid
            plsc.store_scatter(oidx2d, [gvec, iota + half * LANES], orow)
        return carry

    lax.fori_loop(0, NCHUNK, group_body, 0)

    def gather(g, b):
        pltpu.async_copy(wtab_hbm.at[gidx_all.at[pl.ds(g * C, C)]],
                         rows_v.at[b], gsems.at[b])

    gather(0, 0)
    gather(1, 1)

    pltpu.sync_copy(ttab_hbm, tt_v)
    pltpu.sync_copy(stab_hbm, st_v)

    def comb_body(j, carry):
        sl = pl.ds(j * LANES, LANES)
        for t in range(3):
            tv = tt_v[t, sl]
            for s in range(3):
                comb_v[pl.ds((t * 3 + s) * D + j * LANES, LANES)] = (
                    tv + st_v[s, sl]) * INV_SQRT_D
        return carry

    lax.fori_loop(0, DJ, comb_body, 0)

    def finish(g, b):
        pltpu.make_async_copy(wtab_hbm.at[gidx_all.at[pl.ds(g * C, C)]],
                              rows_v.at[b], gsems.at[b]).wait()
        ends = cnts1d[pl.ds(g * LANES, LANES)]
        lo = jnp.int32(0)
        for k in range(NCLS):
            hi = ends[k]
            cvs = [comb_v[pl.ds(k * D + j * LANES, LANES)] for j in range(DJ)]

            def row_body(r, cc):
                for j in range(DJ):
                    plsc.addupdate(rows_v.at[b, r, pl.ds(j * LANES, LANES)],
                                   cvs[j])
                return cc

            lax.fori_loop(lo, hi, row_body, 0)
            lo = hi
        pltpu.async_copy(rows_v.at[b], out_hbm.at[oidx2d.at[g]], osems.at[b])

    def slot_body(s, carry):
        b = lax.rem(s, NBUF)
        finish(s, b)

        @pl.when(s + 2 < NCHUNK)
        def _():
            g2 = s + 2
            b2 = lax.rem(g2, NBUF)

            @pl.when(g2 >= NBUF)
            def _():
                pltpu.make_async_copy(rows_v.at[b2],
                                      out_hbm.at[oidx2d.at[g2 - NBUF]],
                                      osems.at[b2]).wait()

            gather(g2, b2)
        return carry

    lax.fori_loop(0, NCHUNK, slot_body, 0)

    for b in range(NBUF):
        pltpu.make_async_copy(
            rows_v.at[b], out_hbm.at[oidx2d.at[NCHUNK - NBUF + b]],
            osems.at[b]).wait()


@jax.jit
def _run(widx, tidx, sidx, wtab, ttab, stab):
    mesh = plsc.VectorSubcoreMesh(core_axis_name="c", subcore_axis_name="s")
    return pl.kernel(
        _body,
        out_type=jax.ShapeDtypeStruct((N, D), jnp.float32),
        mesh=mesh,
        compiler_params=pltpu.CompilerParams(needs_layout_passes=False),
        scratch_types=[
            pltpu.VMEM((PER_W,), jnp.int32),
            pltpu.VMEM((PER_W,), jnp.int32),
            pltpu.VMEM((PER_W,), jnp.int32),
            pltpu.VMEM((PER_W,), jnp.int32),
            pltpu.VMEM((PER_W,), jnp.int32),
            pltpu.VMEM((NCHUNK, C), jnp.int32),
            pltpu.VMEM((NCHUNK * LANES,), jnp.int32),
            pltpu.VMEM((3, D), jnp.float32),
            pltpu.VMEM((3, D), jnp.float32),
            pltpu.VMEM((NCLS * D,), jnp.float32),
            pltpu.VMEM((NBUF, C, D), jnp.float32),
            pltpu.SemaphoreType.DMA((NBUF,)),
            pltpu.SemaphoreType.DMA((NBUF,)),
        ],
    )(widx, tidx, sidx, wtab, ttab, stab)


def kernel(word_input, position_input, task_input, segment_input,
           word_table, task_table, segment_table):
    del position_input
    B, S = word_input.shape
    widx = word_input.reshape(-1).astype(jnp.int32)
    tidx = task_input.reshape(-1).astype(jnp.int32)
    sidx = segment_input.reshape(-1).astype(jnp.int32)
    out = _run(widx, tidx, sidx, word_table, task_table, segment_table)
    return out.reshape(B, S, D)


# speedup vs baseline: 1.0048x; 1.0048x over previous
import functools
import math

import jax
import jax.numpy as jnp
from jax import lax
from jax.experimental import pallas as pl
from jax.experimental.pallas import tpu as pltpu
from jax.experimental.pallas import tpu_sc as plsc

VOCAB = 50265
D = 768
LANES = 16
DJ = D // LANES
NC = 2
NS = 16
NW = NC * NS
INV_SQRT_D = 1.0 / math.sqrt(D)
NCLS = 9

N = 4 * 4096
PER_W = N // NW
C = 32
NBUF = 4
NCHUNK = PER_W // C


def _body(widx_hbm, tidx_hbm, sidx_hbm, wtab_hbm, ttab_hbm, stab_hbm, out_hbm,
          widx_all, cidx_all, tvec_all, svec_all, gidx_all, oidx2d, cnts1d,
          tt_v, st_v, comb_v, rows_v, gsems, osems):
    wid = lax.axis_index("s") * NC + lax.axis_index("c")
    base = wid * PER_W
    iota = lax.iota(jnp.int32, LANES)

    pltpu.sync_copy(widx_hbm.at[pl.ds(base, PER_W)], widx_all)
    pltpu.sync_copy(tidx_hbm.at[pl.ds(base, PER_W)], tvec_all)
    pltpu.sync_copy(sidx_hbm.at[pl.ds(base, PER_W)], svec_all)

    @plsc.parallel_loop(0, PER_W // LANES, unroll=4)
    def _(j):
        sl = pl.ds(j * LANES, LANES)
        cidx_all[sl] = tvec_all[sl] * 3 + svec_all[sl]

    def group_body(g, carry):
        goff = g * C
        gvec = jnp.full((LANES,), g, jnp.int32)
        k0 = cidx_all[pl.ds(goff, LANES)] * C + iota
        k1 = cidx_all[pl.ds(goff + LANES, LANES)] * C + (iota + LANES)
        s0 = jnp.sort(k0)
        r1 = lax.rev(jnp.sort(k1), (0,))
        halves = (jnp.sort(jnp.minimum(s0, r1)),
                  jnp.sort(jnp.maximum(s0, r1)))
        end = jnp.int32(0)
        endv = jnp.zeros((LANES,), jnp.int32)
        for k in range(NCLS):
            nk = jnp.int32(0)
            for sv in halves:
                eq = lax.shift_right_logical(sv, 5) == k
                nk = nk + plsc.all_reduce_population_count(eq)[0]
            end = end + nk
            endv = endv + jnp.where(iota == k, end, 0)
        cnts1d[pl.ds(g * LANES, LANES)] = endv
        for half in range(2):
            rid = lax.bitwise_and(halves[half], C - 1)
            wv = plsc.load_gather(widx_all, [goff + rid])
            gidx_all[pl.ds(goff + half * LANES, LANES)] = wv
            orow = base + goff + rid
            plsc.store_scatter(oidx2d, [gvec, iota + half * LANES], orow)
        return carry

    def gather(g, b):
        pltpu.async_copy(wtab_hbm.at[gidx_all.at[pl.ds(g * C, C)]],
                         rows_v.at[b], gsems.at[b])

    group_body(0, 0)
    gather(0, 0)
    group_body(1, 0)
    gather(1, 1)
    lax.fori_loop(2, NCHUNK, group_body, 0)

    pltpu.sync_copy(ttab_hbm, tt_v)
    pltpu.sync_copy(stab_hbm, st_v)

    def comb_body(j, carry):
        sl = pl.ds(j * LANES, LANES)
        for t in range(3):
            tv = tt_v[t, sl]
            for s in range(3):
                comb_v[pl.ds((t * 3 + s) * D + j * LANES, LANES)] = (
                    tv + st_v[s, sl]) * INV_SQRT_D
        return carry

    lax.fori_loop(0, DJ, comb_body, 0)

    def finish(g, b):
        pltpu.make_async_copy(wtab_hbm.at[gidx_all.at[pl.ds(g * C, C)]],
                              rows_v.at[b], gsems.at[b]).wait()
        ends = cnts1d[pl.ds(g * LANES, LANES)]
        lo = jnp.int32(0)
        for k in range(NCLS):
            hi = ends[k]
            cvs = [comb_v[pl.ds(k * D + j * LANES, LANES)] for j in range(DJ)]

            def row_body(r, cc):
                for j in range(DJ):
                    plsc.addupdate(rows_v.at[b, r, pl.ds(j * LANES, LANES)],
                                   cvs[j])
                return cc

            lax.fori_loop(lo, hi, row_body, 0)
            lo = hi
        pltpu.async_copy(rows_v.at[b], out_hbm.at[oidx2d.at[g]], osems.at[b])

    def slot_body(s, carry):
        b = lax.rem(s, NBUF)
        finish(s, b)

        @pl.when(s + 2 < NCHUNK)
        def _():
            g2 = s + 2
            b2 = lax.rem(g2, NBUF)

            @pl.when(g2 >= NBUF)
            def _():
                pltpu.make_async_copy(rows_v.at[b2],
                                      out_hbm.at[oidx2d.at[g2 - NBUF]],
                                      osems.at[b2]).wait()

            gather(g2, b2)
        return carry

    lax.fori_loop(0, NCHUNK, slot_body, 0)

    for b in range(NBUF):
        pltpu.make_async_copy(
            rows_v.at[b], out_hbm.at[oidx2d.at[NCHUNK - NBUF + b]],
            osems.at[b]).wait()


@jax.jit
def _run(widx, tidx, sidx, wtab, ttab, stab):
    mesh = plsc.VectorSubcoreMesh(core_axis_name="c", subcore_axis_name="s")
    return pl.kernel(
        _body,
        out_type=jax.ShapeDtypeStruct((N, D), jnp.float32),
        mesh=mesh,
        compiler_params=pltpu.CompilerParams(needs_layout_passes=False),
        scratch_types=[
            pltpu.VMEM((PER_W,), jnp.int32),
            pltpu.VMEM((PER_W,), jnp.int32),
            pltpu.VMEM((PER_W,), jnp.int32),
            pltpu.VMEM((PER_W,), jnp.int32),
            pltpu.VMEM((PER_W,), jnp.int32),
            pltpu.VMEM((NCHUNK, C), jnp.int32),
            pltpu.VMEM((NCHUNK * LANES,), jnp.int32),
            pltpu.VMEM((3, D), jnp.float32),
            pltpu.VMEM((3, D), jnp.float32),
            pltpu.VMEM((NCLS * D,), jnp.float32),
            pltpu.VMEM((NBUF, C, D), jnp.float32),
            pltpu.SemaphoreType.DMA((NBUF,)),
            pltpu.SemaphoreType.DMA((NBUF,)),
        ],
    )(widx, tidx, sidx, wtab, ttab, stab)


def kernel(word_input, position_input, task_input, segment_input,
           word_table, task_table, segment_table):
    del position_input
    B, S = word_input.shape
    widx = word_input.reshape(-1).astype(jnp.int32)
    tidx = task_input.reshape(-1).astype(jnp.int32)
    sidx = segment_input.reshape(-1).astype(jnp.int32)
    out = _run(widx, tidx, sidx, word_table, task_table, segment_table)
    return out.reshape(B, S, D)
